# baseline (speedup 1.0000x reference)
import functools

import jax
import jax.numpy as jnp
from jax import lax
from jax.experimental import pallas as pl
from jax.experimental.pallas import tpu as pltpu
from jax.experimental.pallas import tpu_sc as plsc

D_MODEL = 1024
TIME_STEPS = 8192
BATCH = 16384

_info = plsc.get_sparse_core_info()
_NC = _info.num_cores
_NS = _info.num_subcores
_NW = _NC * _NS
_BPW = BATCH // _NW
_VEC = 16
_NGRP = _BPW // _VEC
_LAG = 2

_mesh = plsc.VectorSubcoreMesh(core_axis_name="c", subcore_axis_name="s")


@functools.partial(
    pl.kernel,
    mesh=_mesh,
    out_type=jax.ShapeDtypeStruct((BATCH, D_MODEL), jnp.float32),
    scratch_types=[
        pltpu.VMEM((_BPW,), jnp.int32),
        pltpu.SemaphoreType.DMA,
    ],
)
def _gather_kernel(pe_hbm, t_hbm, out_hbm, idx_v, sem):
    wid = lax.axis_index("s") * _NC + lax.axis_index("c")
    base = wid * _BPW
    pltpu.sync_copy(t_hbm.at[pl.ds(base, _BPW)], idx_v)

    def issue_group(g):
        v = idx_v[pl.ds(g * _VEC, _VEC)]
        for j in range(_VEC):
            pltpu.async_copy(
                pe_hbm.at[pl.ds(v[j], 1)],
                out_hbm.at[pl.ds(base + g * _VEC + j, 1)],
                sem,
            )

    def drain_group():
        for _ in range(_VEC):
            pltpu.make_async_copy(
                pe_hbm.at[pl.ds(0, 1)], out_hbm.at[pl.ds(base, 1)], sem
            ).wait()

    for g in range(_LAG):
        issue_group(g)

    def body(g, carry):
        del carry
        issue_group(g)
        drain_group()
        return 0

    lax.fori_loop(_LAG, _NGRP, body, 0)

    for _ in range(_LAG):
        drain_group()


def kernel(pe, t):
    return _gather_kernel(pe, t)


# speedup vs baseline: 30.5109x; 30.5109x over previous
import functools

import jax
import jax.numpy as jnp
from jax import lax
from jax.experimental import pallas as pl
from jax.experimental.pallas import tpu as pltpu
from jax.experimental.pallas import tpu_sc as plsc

D_MODEL = 1024
TIME_STEPS = 8192
BATCH = 16384

_info = plsc.get_sparse_core_info()
_NC = _info.num_cores
_NS = _info.num_subcores
_NW = _NC * _NS
_BPW = BATCH // _NW
_CHUNK = 16
_NCHUNK = _BPW // _CHUNK
_NBUF = 6

_mesh = plsc.VectorSubcoreMesh(core_axis_name="c", subcore_axis_name="s")


@functools.partial(
    pl.kernel,
    mesh=_mesh,
    out_type=jax.ShapeDtypeStruct((BATCH, D_MODEL), jnp.float32),
    scratch_types=[
        pltpu.VMEM((_BPW,), jnp.int32),
    ]
    + [pltpu.VMEM((_CHUNK, D_MODEL), jnp.float32) for _ in range(_NBUF)]
    + [pltpu.SemaphoreType.DMA for _ in range(2 * _NBUF)],
)
def _gather_kernel(pe_hbm, t_hbm, out_hbm, idx_v, *bufs):
    rows = bufs[:_NBUF]
    gsem = bufs[_NBUF : 2 * _NBUF]
    wsem = bufs[2 * _NBUF :]
    wid = lax.axis_index("s") * _NC + lax.axis_index("c")
    base = wid * _BPW
    pltpu.sync_copy(t_hbm.at[pl.ds(base, _BPW)], idx_v)

    def gather(c):
        b = c % _NBUF
        idx_slice = idx_v.at[pl.ds(c * _CHUNK, _CHUNK)]
        return pltpu.async_copy(pe_hbm.at[idx_slice], rows[b], gsem[b])

    def writeback(c):
        b = c % _NBUF
        dst = out_hbm.at[pl.ds(base + c * _CHUNK, _CHUNK)]
        return pltpu.async_copy(rows[b], dst, wsem[b])

    g = [None] * _NBUF
    w = [None] * _NBUF
    for c in range(_NBUF - 1):
        g[c % _NBUF] = gather(c)
    for c in range(_NCHUNK):
        b = c % _NBUF
        nxt = c + _NBUF - 1
        if nxt < _NCHUNK:
            nb = nxt % _NBUF
            if w[nb] is not None:
                w[nb].wait()
                w[nb] = None
            g[nb] = gather(nxt)
        g[b].wait()
        w[b] = writeback(c)
    for b in range(_NBUF):
        if w[b] is not None:
            w[b].wait()


def kernel(pe, t):
    return _gather_kernel(pe, t)
